# baseline (speedup 1.0000x reference)
import jax
import jax.numpy as jnp
from jax.experimental import pallas as pl
from jax.experimental.pallas import tpu as pltpu

HI = jax.lax.Precision.HIGHEST
DIM = 1536
SEQLEN = 896
E = 8
TOPK = 3
TRACKS = 1643
BOUNDS = [(0, 228), (228, 519), (519, 1286), (1286, 1643)]
TT = [e - s for (s, e) in BOUNDS]
B = 2
NEG = -1e30

_CT0 = (((0,), (0,)), ((), ()))
_CT11 = (((1,), (1,)), ((), ()))


def _prologue_kernel(x_ref, wgs_ref, bgs_ref, emb_ref, wegs_ref, begs_ref,
                     swap_ref, w_out_ref, zp_ref, acc_ref):
    k = pl.program_id(0)

    @pl.when(k == 0)
    def _():
        acc_ref[...] = jnp.zeros_like(acc_ref)

    acc_ref[...] += jnp.sum(x_ref[...], axis=1)

    @pl.when(k == pl.num_programs(0) - 1)
    def _():
        pooled = acc_ref[...] / SEQLEN
        gl = jnp.dot(pooled.astype(jnp.bfloat16), wgs_ref[...],
                     preferred_element_type=jnp.float32) + bgs_ref[...]
        el = jnp.dot(emb_ref[...].astype(jnp.bfloat16), wegs_ref[...],
                     preferred_element_type=jnp.float32) + begs_ref[...]
        glp = jnp.dot(gl, swap_ref[...], precision=HI,
                      preferred_element_type=jnp.float32)
        elp = jnp.dot(el, swap_ref[...], precision=HI,
                      preferred_element_type=jnp.float32)
        m1 = jnp.maximum(gl, glp)
        tw = jnp.exp(gl - m1) / (jnp.exp(gl - m1) + jnp.exp(glp - m1))
        m2 = jnp.maximum(el, elp)
        ew = jnp.exp(el - m2) / (jnp.exp(el - m2) + jnp.exp(elp - m2))
        w_out_ref[...] = (tw + ew) / 2.0
        zp_ref[...] = (jnp.sum(gl * gl, keepdims=True) / (2.0 * B)
                       + jnp.sum(el * el, keepdims=True) / 2.0)


def _gating_kernel(x_ref, wgt_ref, bgt_ref, tet_ref, oh4_ref, w_ref,
                   gates_ref, allg_ref, zl_out_ref, cv_out_ref,
                   imp_ref, zl_ref):
    b = pl.program_id(0)

    @pl.when(b == 0)
    def _():
        imp_ref[...] = jnp.zeros_like(imp_ref)
        zl_ref[...] = jnp.zeros_like(zl_ref)
        allg_ref[...] = jnp.zeros_like(allg_ref)

    oh4 = oh4_ref[...]
    te_sel = jax.lax.dot_general(tet_ref[...], oh4, _CT0,
                                 precision=HI,
                                 preferred_element_type=jnp.float32)
    xte = (x_ref[0] + te_sel).astype(jnp.bfloat16)
    logits = jnp.dot(wgt_ref[...], xte,
                     preferred_element_type=jnp.float32)
    logits = logits + bgt_ref[...]
    logits = jnp.where(logits > 0, logits, 0.01 * logits)

    sub = jax.lax.broadcasted_iota(jnp.int32, (E, TRACKS), 0)
    wv = w_ref[...]
    wb = jnp.where(b == 0, wv[0:1, :], wv[1:2, :])
    gates_bt = jnp.zeros((E, TRACKS), jnp.float32)
    for g in range(8):
        lg = logits[8 * g:8 * (g + 1), :]
        ag = oh4[g // 2:g // 2 + 1, :]
        zl_ref[...] += (jnp.sum(ag * lg * lg, keepdims=True)
                        / (2.0 * TT[g // 2] * E))
        cur = lg
        vals, ohs = [], []
        for _k in range(TOPK):
            m = jnp.max(cur, axis=0, keepdims=True)
            idx = jnp.min(jnp.where(cur == m, sub, E), axis=0, keepdims=True)
            oh = (sub == idx).astype(jnp.float32)
            vals.append(m)
            ohs.append(oh)
            cur = jnp.where(oh > 0, NEG, cur)
        es = [jnp.exp(v - vals[0]) for v in vals]
        z = es[0] + es[1] + es[2]
        tg = (es[0] * ohs[0] + es[1] * ohs[1] + es[2] * ohs[2]) / z
        tga = tg * ag
        imp_ref[g:g + 1, :] += jnp.sum(tga, axis=1, keepdims=True).reshape(
            1, E)
        gates_bt = gates_bt + (wb[0:1, g:g + 1] * tga)

    gates_ref[0] = gates_bt
    allg_ref[...] += jax.lax.dot_general(
        oh4, gates_bt, _CT11, precision=HI,
        preferred_element_type=jnp.float32)

    @pl.when(b == pl.num_programs(0) - 1)
    def _():
        imp = imp_ref[...]
        mean = jnp.mean(imp, axis=1, keepdims=True)
        var = jnp.mean((imp - mean) ** 2, axis=1, keepdims=True)
        cv_out_ref[...] = jnp.sum(var / (mean * mean + 1e-10), keepdims=True)
        zl_out_ref[...] = zl_ref[...]


def _expert_kernel(x_ref, w1_ref, b1_ref, w2_ref, b2_ref, gates_ref, out_ref):
    e = pl.program_id(1)
    for lo, hi in ((0, 768), (768, TRACKS)):
        xc = x_ref[0, :, lo:hi]
        gc = gates_ref[0, :, lo:hi]
        h = jax.lax.dot_general(w1_ref[0], xc.astype(jnp.bfloat16), _CT0,
                                preferred_element_type=jnp.float32
                                ) + b1_ref[0]
        h = jnp.maximum(h, 0.0).astype(jnp.bfloat16)
        o = jax.lax.dot_general(w2_ref[0], h, _CT0,
                                preferred_element_type=jnp.float32
                                ) + b2_ref[0]
        go = gc * o

        @pl.when(e == 0)
        def _(xc=xc, go=go, lo=lo, hi=hi):
            out_ref[0, :, lo:hi] = xc + go

        @pl.when(e != 0)
        def _(go=go, lo=lo, hi=hi):
            out_ref[0, :, lo:hi] += go


def kernel(x, out, embedding, W_gs, b_gs, W_egs, b_egs, Wg, bg, W1, b1, W2,
           b2, te):
    f32 = jnp.float32
    bf16 = jnp.bfloat16
    wgt = jnp.transpose(Wg, (0, 2, 1)).reshape(64, SEQLEN).astype(bf16)
    bgt = bg.reshape(64, 1)
    tet = te
    tids = (jnp.arange(TRACKS)[None, :] >= jnp.array(
        [b[0] for b in BOUNDS])[:, None]).astype(jnp.int32)
    oh4 = (tids.sum(axis=0) - 1 == jnp.arange(4)[:, None]).astype(f32)
    swap = jnp.eye(8, dtype=f32)[jnp.arange(8) ^ 1]
    w1_bf = W1.astype(bf16)
    w2_bf = W2.astype(bf16)

    w_pair, zp = pl.pallas_call(
        _prologue_kernel,
        grid=(8,),
        in_specs=[
            pl.BlockSpec((B, SEQLEN // 8, 2 * DIM), lambda k: (0, k, 0)),
            pl.BlockSpec((2 * DIM, E), lambda k: (0, 0)),
            pl.BlockSpec((1, E), lambda k: (0, 0)),
            pl.BlockSpec((1, DIM), lambda k: (0, 0)),
            pl.BlockSpec((DIM, E), lambda k: (0, 0)),
            pl.BlockSpec((1, E), lambda k: (0, 0)),
            pl.BlockSpec((E, E), lambda k: (0, 0)),
        ],
        out_specs=[
            pl.BlockSpec((B, E), lambda k: (0, 0)),
            pl.BlockSpec((1, 1), lambda k: (0, 0)),
        ],
        out_shape=[
            jax.ShapeDtypeStruct((B, E), f32),
            jax.ShapeDtypeStruct((1, 1), f32),
        ],
        scratch_shapes=[pltpu.VMEM((B, 2 * DIM), f32)],
    )(x, W_gs.astype(bf16), b_gs.reshape(1, E), embedding[:, 0, :],
      W_egs.astype(bf16), b_egs.reshape(1, E), swap)

    gates, all_gates, zl_g, cv_g = pl.pallas_call(
        _gating_kernel,
        grid=(B,),
        in_specs=[
            pl.BlockSpec((1, SEQLEN, TRACKS), lambda b: (b, 0, 0)),
            pl.BlockSpec((64, SEQLEN), lambda b: (0, 0)),
            pl.BlockSpec((64, 1), lambda b: (0, 0)),
            pl.BlockSpec((4, SEQLEN), lambda b: (0, 0)),
            pl.BlockSpec((4, TRACKS), lambda b: (0, 0)),
            pl.BlockSpec((B, E), lambda b: (0, 0)),
        ],
        out_specs=[
            pl.BlockSpec((1, E, TRACKS), lambda b: (b, 0, 0)),
            pl.BlockSpec((4, E), lambda b: (0, 0)),
            pl.BlockSpec((1, 1), lambda b: (0, 0)),
            pl.BlockSpec((1, 1), lambda b: (0, 0)),
        ],
        out_shape=[
            jax.ShapeDtypeStruct((B, E, TRACKS), f32),
            jax.ShapeDtypeStruct((4, E), f32),
            jax.ShapeDtypeStruct((1, 1), f32),
            jax.ShapeDtypeStruct((1, 1), f32),
        ],
        scratch_shapes=[pltpu.VMEM((E, E), f32), pltpu.VMEM((1, 1), f32)],
        compiler_params=pltpu.CompilerParams(
            vmem_limit_bytes=128 * 1024 * 1024),
    )(out, wgt, bgt, tet, oh4, w_pair)

    gates16 = gates.reshape(B * E, 1, TRACKS)

    y = pl.pallas_call(
        _expert_kernel,
        grid=(B, E),
        in_specs=[
            pl.BlockSpec((1, SEQLEN, TRACKS), lambda b, e: (b, 0, 0)),
            pl.BlockSpec((1, SEQLEN, 2 * SEQLEN), lambda b, e: (e, 0, 0)),
            pl.BlockSpec((1, 2 * SEQLEN, 1), lambda b, e: (e, 0, 0)),
            pl.BlockSpec((1, 2 * SEQLEN, SEQLEN), lambda b, e: (e, 0, 0)),
            pl.BlockSpec((1, SEQLEN, 1), lambda b, e: (e, 0, 0)),
            pl.BlockSpec((1, 1, TRACKS), lambda b, e: (8 * b + e, 0, 0)),
        ],
        out_specs=pl.BlockSpec((1, SEQLEN, TRACKS), lambda b, e: (b, 0, 0)),
        out_shape=jax.ShapeDtypeStruct((B, SEQLEN, TRACKS), f32),
        compiler_params=pltpu.CompilerParams(
            vmem_limit_bytes=128 * 1024 * 1024),
    )(out, w1_bf, b1.reshape(E, 2 * SEQLEN, 1), w2_bf,
      b2.reshape(E, SEQLEN, 1), gates16)

    total_zloss = (zl_g[0, 0] + zp[0, 0]).reshape(())
    total_cvloss = cv_g[0, 0].reshape(())
    return (y, all_gates, total_zloss, total_cvloss, w_pair)


# speedup vs baseline: 1.0156x; 1.0156x over previous
import jax
import jax.numpy as jnp
from jax.experimental import pallas as pl
from jax.experimental.pallas import tpu as pltpu

HI = jax.lax.Precision.HIGHEST
DIM = 1536
SEQLEN = 896
E = 8
TOPK = 3
TRACKS = 1643
BOUNDS = [(0, 228), (228, 519), (519, 1286), (1286, 1643)]
TT = [e - s for (s, e) in BOUNDS]
B = 2
NEG = -1e30

_CT0 = (((0,), (0,)), ((), ()))
_CT11 = (((1,), (1,)), ((), ()))


def _prologue_kernel(x_ref, wgs_ref, bgs_ref, emb_ref, wegs_ref, begs_ref,
                     swap_ref, w_out_ref, zp_ref, acc_ref):
    k = pl.program_id(0)

    @pl.when(k == 0)
    def _():
        acc_ref[...] = jnp.zeros_like(acc_ref)

    acc_ref[...] += jnp.sum(x_ref[...], axis=1)

    @pl.when(k == pl.num_programs(0) - 1)
    def _():
        pooled = acc_ref[...] / SEQLEN
        gl = jnp.dot(pooled.astype(jnp.bfloat16), wgs_ref[...],
                     preferred_element_type=jnp.float32) + bgs_ref[...]
        el = jnp.dot(emb_ref[...].astype(jnp.bfloat16), wegs_ref[...],
                     preferred_element_type=jnp.float32) + begs_ref[...]
        glp = jnp.dot(gl, swap_ref[...], precision=HI,
                      preferred_element_type=jnp.float32)
        elp = jnp.dot(el, swap_ref[...], precision=HI,
                      preferred_element_type=jnp.float32)
        m1 = jnp.maximum(gl, glp)
        tw = jnp.exp(gl - m1) / (jnp.exp(gl - m1) + jnp.exp(glp - m1))
        m2 = jnp.maximum(el, elp)
        ew = jnp.exp(el - m2) / (jnp.exp(el - m2) + jnp.exp(elp - m2))
        w_out_ref[...] = (tw + ew) / 2.0
        zp_ref[...] = (jnp.sum(gl * gl, keepdims=True) / (2.0 * B)
                       + jnp.sum(el * el, keepdims=True) / 2.0)


def _gating_kernel(x_ref, wgt_ref, bgt_ref, tet_ref, oh4_ref, w_ref,
                   gates_ref, allg_ref, zl_out_ref, cv_out_ref,
                   imp_ref, zl_ref):
    b = pl.program_id(0)

    @pl.when(b == 0)
    def _():
        imp_ref[...] = jnp.zeros_like(imp_ref)
        zl_ref[...] = jnp.zeros_like(zl_ref)
        allg_ref[...] = jnp.zeros_like(allg_ref)

    oh4 = oh4_ref[...]
    te_sel = jax.lax.dot_general(tet_ref[...], oh4, _CT0,
                                 precision=HI,
                                 preferred_element_type=jnp.float32)
    xte = (x_ref[0] + te_sel).astype(jnp.bfloat16)
    logits = jnp.dot(wgt_ref[...], xte,
                     preferred_element_type=jnp.float32)
    logits = logits + bgt_ref[...]
    logits = jnp.where(logits > 0, logits, 0.01 * logits)

    sub = jax.lax.broadcasted_iota(jnp.int32, (E, TRACKS), 0)
    wv = w_ref[...]
    wb = jnp.where(b == 0, wv[0:1, :], wv[1:2, :])
    gates_bt = jnp.zeros((E, TRACKS), jnp.float32)
    for g in range(8):
        lg = logits[8 * g:8 * (g + 1), :]
        ag = oh4[g // 2:g // 2 + 1, :]
        zl_ref[...] += (jnp.sum(ag * lg * lg, keepdims=True)
                        / (2.0 * TT[g // 2] * E))
        cur = lg
        vals, ohs = [], []
        for _k in range(TOPK):
            m = jnp.max(cur, axis=0, keepdims=True)
            idx = jnp.min(jnp.where(cur == m, sub, E), axis=0, keepdims=True)
            oh = (sub == idx).astype(jnp.float32)
            vals.append(m)
            ohs.append(oh)
            cur = jnp.where(oh > 0, NEG, cur)
        es = [jnp.exp(v - vals[0]) for v in vals]
        z = es[0] + es[1] + es[2]
        tg = (es[0] * ohs[0] + es[1] * ohs[1] + es[2] * ohs[2]) / z
        tga = tg * ag
        imp_ref[g:g + 1, :] += jnp.sum(tga, axis=1, keepdims=True).reshape(
            1, E)
        gates_bt = gates_bt + (wb[0:1, g:g + 1] * tga)

    gates_ref[0] = gates_bt
    allg_ref[...] += jax.lax.dot_general(
        oh4, gates_bt, _CT11, precision=HI,
        preferred_element_type=jnp.float32)

    @pl.when(b == pl.num_programs(0) - 1)
    def _():
        imp = imp_ref[...]
        mean = jnp.mean(imp, axis=1, keepdims=True)
        var = jnp.mean((imp - mean) ** 2, axis=1, keepdims=True)
        cv_out_ref[...] = jnp.sum(var / (mean * mean + 1e-10), keepdims=True)
        zl_out_ref[...] = zl_ref[...]


def _expert_kernel(x_ref, w1_ref, b1_ref, w2_ref, b2_ref, gates_ref, out_ref):
    e = pl.program_id(0)
    for b in range(B):
        xs = x_ref[b]
        g = gates_ref[b, pl.ds(e, 1), :]
        h = jax.lax.dot_general(w1_ref[0], xs.astype(jnp.bfloat16), _CT0,
                                preferred_element_type=jnp.float32
                                ) + b1_ref[0]
        h = jnp.maximum(h, 0.0).astype(jnp.bfloat16)
        o = jax.lax.dot_general(w2_ref[0], h, _CT0,
                                preferred_element_type=jnp.float32
                                ) + b2_ref[0]
        go = g * o

        @pl.when(e == 0)
        def _(b=b, xs=xs, go=go):
            out_ref[b] = xs + go

        @pl.when(e != 0)
        def _(b=b, go=go):
            out_ref[b] += go


def kernel(x, out, embedding, W_gs, b_gs, W_egs, b_egs, Wg, bg, W1, b1, W2,
           b2, te):
    f32 = jnp.float32
    bf16 = jnp.bfloat16
    wgt = jnp.transpose(Wg, (0, 2, 1)).reshape(64, SEQLEN).astype(bf16)
    bgt = bg.reshape(64, 1)
    tet = te
    tids = (jnp.arange(TRACKS)[None, :] >= jnp.array(
        [b[0] for b in BOUNDS])[:, None]).astype(jnp.int32)
    oh4 = (tids.sum(axis=0) - 1 == jnp.arange(4)[:, None]).astype(f32)
    swap = jnp.eye(8, dtype=f32)[jnp.arange(8) ^ 1]
    w1_bf = W1.astype(bf16)
    w2_bf = W2.astype(bf16)

    w_pair, zp = pl.pallas_call(
        _prologue_kernel,
        grid=(8,),
        in_specs=[
            pl.BlockSpec((B, SEQLEN // 8, 2 * DIM), lambda k: (0, k, 0)),
            pl.BlockSpec((2 * DIM, E), lambda k: (0, 0)),
            pl.BlockSpec((1, E), lambda k: (0, 0)),
            pl.BlockSpec((1, DIM), lambda k: (0, 0)),
            pl.BlockSpec((DIM, E), lambda k: (0, 0)),
            pl.BlockSpec((1, E), lambda k: (0, 0)),
            pl.BlockSpec((E, E), lambda k: (0, 0)),
        ],
        out_specs=[
            pl.BlockSpec((B, E), lambda k: (0, 0)),
            pl.BlockSpec((1, 1), lambda k: (0, 0)),
        ],
        out_shape=[
            jax.ShapeDtypeStruct((B, E), f32),
            jax.ShapeDtypeStruct((1, 1), f32),
        ],
        scratch_shapes=[pltpu.VMEM((B, 2 * DIM), f32)],
    )(x, W_gs.astype(bf16), b_gs.reshape(1, E), embedding[:, 0, :],
      W_egs.astype(bf16), b_egs.reshape(1, E), swap)

    gates, all_gates, zl_g, cv_g = pl.pallas_call(
        _gating_kernel,
        grid=(B,),
        in_specs=[
            pl.BlockSpec((1, SEQLEN, TRACKS), lambda b: (b, 0, 0)),
            pl.BlockSpec((64, SEQLEN), lambda b: (0, 0)),
            pl.BlockSpec((64, 1), lambda b: (0, 0)),
            pl.BlockSpec((4, SEQLEN), lambda b: (0, 0)),
            pl.BlockSpec((4, TRACKS), lambda b: (0, 0)),
            pl.BlockSpec((B, E), lambda b: (0, 0)),
        ],
        out_specs=[
            pl.BlockSpec((1, E, TRACKS), lambda b: (b, 0, 0)),
            pl.BlockSpec((4, E), lambda b: (0, 0)),
            pl.BlockSpec((1, 1), lambda b: (0, 0)),
            pl.BlockSpec((1, 1), lambda b: (0, 0)),
        ],
        out_shape=[
            jax.ShapeDtypeStruct((B, E, TRACKS), f32),
            jax.ShapeDtypeStruct((4, E), f32),
            jax.ShapeDtypeStruct((1, 1), f32),
            jax.ShapeDtypeStruct((1, 1), f32),
        ],
        scratch_shapes=[pltpu.VMEM((E, E), f32), pltpu.VMEM((1, 1), f32)],
        compiler_params=pltpu.CompilerParams(
            vmem_limit_bytes=128 * 1024 * 1024),
    )(out, wgt, bgt, tet, oh4, w_pair)

    y = pl.pallas_call(
        _expert_kernel,
        grid=(E,),
        in_specs=[
            pl.BlockSpec((B, SEQLEN, TRACKS), lambda e: (0, 0, 0)),
            pl.BlockSpec((1, SEQLEN, 2 * SEQLEN), lambda e: (e, 0, 0)),
            pl.BlockSpec((1, 2 * SEQLEN, 1), lambda e: (e, 0, 0)),
            pl.BlockSpec((1, 2 * SEQLEN, SEQLEN), lambda e: (e, 0, 0)),
            pl.BlockSpec((1, SEQLEN, 1), lambda e: (e, 0, 0)),
            pl.BlockSpec((B, E, TRACKS), lambda e: (0, 0, 0)),
        ],
        out_specs=pl.BlockSpec((B, SEQLEN, TRACKS), lambda e: (0, 0, 0)),
        out_shape=jax.ShapeDtypeStruct((B, SEQLEN, TRACKS), f32),
        compiler_params=pltpu.CompilerParams(
            vmem_limit_bytes=128 * 1024 * 1024),
    )(out, w1_bf, b1.reshape(E, 2 * SEQLEN, 1), w2_bf,
      b2.reshape(E, SEQLEN, 1), gates)

    total_zloss = (zl_g[0, 0] + zp[0, 0]).reshape(())
    total_cvloss = cv_g[0, 0].reshape(())
    return (y, all_gates, total_zloss, total_cvloss, w_pair)


# speedup vs baseline: 1.0316x; 1.0158x over previous
import jax
import jax.numpy as jnp
from jax.experimental import pallas as pl
from jax.experimental.pallas import tpu as pltpu

HI = jax.lax.Precision.HIGHEST
DIM = 1536
SEQLEN = 896
E = 8
TOPK = 3
TRACKS = 1643
BOUNDS = [(0, 228), (228, 519), (519, 1286), (1286, 1643)]
TT = [e - s for (s, e) in BOUNDS]
B = 2
NEG = -1e30

_CT0 = (((0,), (0,)), ((), ()))
_CT11 = (((1,), (1,)), ((), ()))


def _prologue_kernel(x_ref, wgs_ref, bgs_ref, emb_ref, wegs_ref, begs_ref,
                     swap_ref, w_out_ref, zp_ref, acc_ref):
    k = pl.program_id(0)

    @pl.when(k == 0)
    def _():
        acc_ref[...] = jnp.zeros_like(acc_ref)

    acc_ref[...] += jnp.sum(x_ref[...], axis=1)

    @pl.when(k == pl.num_programs(0) - 1)
    def _():
        pooled = acc_ref[...] / SEQLEN
        gl = jnp.dot(pooled.astype(jnp.bfloat16), wgs_ref[...],
                     preferred_element_type=jnp.float32) + bgs_ref[...]
        el = jnp.dot(emb_ref[...].astype(jnp.bfloat16), wegs_ref[...],
                     preferred_element_type=jnp.float32) + begs_ref[...]
        glp = jnp.dot(gl, swap_ref[...], precision=HI,
                      preferred_element_type=jnp.float32)
        elp = jnp.dot(el, swap_ref[...], precision=HI,
                      preferred_element_type=jnp.float32)
        m1 = jnp.maximum(gl, glp)
        tw = jnp.exp(gl - m1) / (jnp.exp(gl - m1) + jnp.exp(glp - m1))
        m2 = jnp.maximum(el, elp)
        ew = jnp.exp(el - m2) / (jnp.exp(el - m2) + jnp.exp(elp - m2))
        w_out_ref[...] = (tw + ew) / 2.0
        zp_ref[...] = (jnp.sum(gl * gl, keepdims=True) / (2.0 * B)
                       + jnp.sum(el * el, keepdims=True) / 2.0)


def _gating_kernel(x_ref, wgt_ref, bgt_ref, tet_ref, oh4_ref, w_ref,
                   gates_ref, allg_ref, zl_out_ref, cv_out_ref,
                   imp_ref, zl_ref):
    b = pl.program_id(0)

    @pl.when(b == 0)
    def _():
        imp_ref[...] = jnp.zeros_like(imp_ref)
        zl_ref[...] = jnp.zeros_like(zl_ref)
        allg_ref[...] = jnp.zeros_like(allg_ref)

    oh4 = oh4_ref[...]
    te_sel = jax.lax.dot_general(tet_ref[...], oh4, _CT0,
                                 precision=HI,
                                 preferred_element_type=jnp.float32)
    xte = (x_ref[0] + te_sel).astype(jnp.bfloat16)
    logits = jnp.dot(wgt_ref[...], xte,
                     preferred_element_type=jnp.float32)
    logits = logits + bgt_ref[...]
    logits = jnp.where(logits > 0, logits, 0.01 * logits)

    sub = jax.lax.broadcasted_iota(jnp.int32, (E, TRACKS), 0)
    wv = w_ref[...]
    wb = jnp.where(b == 0, wv[0:1, :], wv[1:2, :])
    gates_bt = jnp.zeros((E, TRACKS), jnp.float32)
    for g in range(8):
        lg = logits[8 * g:8 * (g + 1), :]
        ag = oh4[g // 2:g // 2 + 1, :]
        zl_ref[...] += (jnp.sum(ag * lg * lg, keepdims=True)
                        / (2.0 * TT[g // 2] * E))
        cur = lg
        vals, ohs = [], []
        for _k in range(TOPK):
            m = jnp.max(cur, axis=0, keepdims=True)
            idx = jnp.min(jnp.where(cur == m, sub, E), axis=0, keepdims=True)
            oh = (sub == idx).astype(jnp.float32)
            vals.append(m)
            ohs.append(oh)
            cur = jnp.where(oh > 0, NEG, cur)
        es = [jnp.exp(v - vals[0]) for v in vals]
        z = es[0] + es[1] + es[2]
        tg = (es[0] * ohs[0] + es[1] * ohs[1] + es[2] * ohs[2]) / z
        tga = tg * ag
        imp_ref[g:g + 1, :] += jnp.sum(tga, axis=1, keepdims=True).reshape(
            1, E)
        gates_bt = gates_bt + (wb[0:1, g:g + 1] * tga)

    gates_ref[0] = gates_bt
    allg_ref[...] += jax.lax.dot_general(
        oh4, gates_bt, _CT11, precision=HI,
        preferred_element_type=jnp.float32)

    @pl.when(b == pl.num_programs(0) - 1)
    def _():
        imp = imp_ref[...]
        mean = jnp.mean(imp, axis=1, keepdims=True)
        var = jnp.mean((imp - mean) ** 2, axis=1, keepdims=True)
        cv_out_ref[...] = jnp.sum(var / (mean * mean + 1e-10), keepdims=True)
        zl_out_ref[...] = zl_ref[...]


def _expert_kernel(x_ref, w1_ref, b1_ref, w2_ref, b2_ref, gates_ref, out_ref):
    e = pl.program_id(1)
    xs = x_ref[0]
    h = jax.lax.dot_general(w1_ref[0], xs.astype(jnp.bfloat16), _CT0,
                            preferred_element_type=jnp.float32) + b1_ref[0]
    h = jnp.maximum(h, 0.0).astype(jnp.bfloat16)
    o = jax.lax.dot_general(w2_ref[0], h, _CT0,
                            preferred_element_type=jnp.float32) + b2_ref[0]
    g = gates_ref[0]

    @pl.when(e == 0)
    def _():
        out_ref[0] = xs + g * o

    @pl.when(e != 0)
    def _():
        out_ref[0] += g * o


def kernel(x, out, embedding, W_gs, b_gs, W_egs, b_egs, Wg, bg, W1, b1, W2,
           b2, te):
    f32 = jnp.float32
    bf16 = jnp.bfloat16
    wgt = jnp.transpose(Wg, (0, 2, 1)).reshape(64, SEQLEN).astype(bf16)
    bgt = bg.reshape(64, 1)
    tet = te
    tids = (jnp.arange(TRACKS)[None, :] >= jnp.array(
        [b[0] for b in BOUNDS])[:, None]).astype(jnp.int32)
    oh4 = (tids.sum(axis=0) - 1 == jnp.arange(4)[:, None]).astype(f32)
    swap = jnp.eye(8, dtype=f32)[jnp.arange(8) ^ 1]
    w1_bf = W1.astype(bf16)
    w2_bf = W2.astype(bf16)

    w_pair, zp = pl.pallas_call(
        _prologue_kernel,
        grid=(8,),
        in_specs=[
            pl.BlockSpec((B, SEQLEN // 8, 2 * DIM), lambda k: (0, k, 0)),
            pl.BlockSpec((2 * DIM, E), lambda k: (0, 0)),
            pl.BlockSpec((1, E), lambda k: (0, 0)),
            pl.BlockSpec((1, DIM), lambda k: (0, 0)),
            pl.BlockSpec((DIM, E), lambda k: (0, 0)),
            pl.BlockSpec((1, E), lambda k: (0, 0)),
            pl.BlockSpec((E, E), lambda k: (0, 0)),
        ],
        out_specs=[
            pl.BlockSpec((B, E), lambda k: (0, 0)),
            pl.BlockSpec((1, 1), lambda k: (0, 0)),
        ],
        out_shape=[
            jax.ShapeDtypeStruct((B, E), f32),
            jax.ShapeDtypeStruct((1, 1), f32),
        ],
        scratch_shapes=[pltpu.VMEM((B, 2 * DIM), f32)],
    )(x, W_gs.astype(bf16), b_gs.reshape(1, E), embedding[:, 0, :],
      W_egs.astype(bf16), b_egs.reshape(1, E), swap)

    gates, all_gates, zl_g, cv_g = pl.pallas_call(
        _gating_kernel,
        grid=(B,),
        in_specs=[
            pl.BlockSpec((1, SEQLEN, TRACKS), lambda b: (b, 0, 0)),
            pl.BlockSpec((64, SEQLEN), lambda b: (0, 0)),
            pl.BlockSpec((64, 1), lambda b: (0, 0)),
            pl.BlockSpec((4, SEQLEN), lambda b: (0, 0)),
            pl.BlockSpec((4, TRACKS), lambda b: (0, 0)),
            pl.BlockSpec((B, E), lambda b: (0, 0)),
        ],
        out_specs=[
            pl.BlockSpec((1, E, TRACKS), lambda b: (b, 0, 0)),
            pl.BlockSpec((4, E), lambda b: (0, 0)),
            pl.BlockSpec((1, 1), lambda b: (0, 0)),
            pl.BlockSpec((1, 1), lambda b: (0, 0)),
        ],
        out_shape=[
            jax.ShapeDtypeStruct((B, E, TRACKS), f32),
            jax.ShapeDtypeStruct((4, E), f32),
            jax.ShapeDtypeStruct((1, 1), f32),
            jax.ShapeDtypeStruct((1, 1), f32),
        ],
        scratch_shapes=[pltpu.VMEM((E, E), f32), pltpu.VMEM((1, 1), f32)],
        compiler_params=pltpu.CompilerParams(
            vmem_limit_bytes=128 * 1024 * 1024),
    )(out, wgt, bgt, tet, oh4, w_pair)

    gates16 = gates.reshape(B * E, 1, TRACKS)

    y = pl.pallas_call(
        _expert_kernel,
        grid=(B, E),
        in_specs=[
            pl.BlockSpec((1, SEQLEN, TRACKS), lambda b, e: (b, 0, 0)),
            pl.BlockSpec((1, SEQLEN, 2 * SEQLEN), lambda b, e: (e, 0, 0)),
            pl.BlockSpec((1, 2 * SEQLEN, 1), lambda b, e: (e, 0, 0)),
            pl.BlockSpec((1, 2 * SEQLEN, SEQLEN), lambda b, e: (e, 0, 0)),
            pl.BlockSpec((1, SEQLEN, 1), lambda b, e: (e, 0, 0)),
            pl.BlockSpec((1, 1, TRACKS), lambda b, e: (8 * b + e, 0, 0)),
        ],
        out_specs=pl.BlockSpec((1, SEQLEN, TRACKS), lambda b, e: (b, 0, 0)),
        out_shape=jax.ShapeDtypeStruct((B, SEQLEN, TRACKS), f32),
        compiler_params=pltpu.CompilerParams(
            vmem_limit_bytes=128 * 1024 * 1024),
    )(out, w1_bf, b1.reshape(E, 2 * SEQLEN, 1), w2_bf,
      b2.reshape(E, SEQLEN, 1), gates16)

    total_zloss = (zl_g[0, 0] + zp[0, 0]).reshape(())
    total_cvloss = cv_g[0, 0].reshape(())
    return (y, all_gates, total_zloss, total_cvloss, w_pair)
